# Initial kernel scaffold; baseline (speedup 1.0000x reference)
import jax
import jax.numpy as jnp
from jax.experimental import pallas as pl


def kernel(x, ln1_g, ln1_b, ln2_g, ln2_b, Wqkv, bqkv, Wo, bo, Wr, W1, b1, W2, b2):
    raise NotImplementedError("write your pallas kernel here")


# speedup vs baseline: 1.3643x; 1.3643x over previous
import functools
import jax
import jax.numpy as jnp
from jax.experimental import pallas as pl
from jax.experimental.pallas import tpu as pltpu

S, D, H, E, K, F = 2048, 768, 12, 8, 2, 1536
DH = D // H
SB = 256
QB = 512


def _ln_qkv_body(x_ref, g_ref, b_ref, w_ref, bias_ref, qkv_ref):
    x = x_ref[...]
    m = jnp.mean(x, axis=-1, keepdims=True)
    v = jnp.mean((x - m) ** 2, axis=-1, keepdims=True)
    h = (x - m) * jax.lax.rsqrt(v + 1e-5) * g_ref[...] + b_ref[...]
    qkv_ref[...] = (
        jnp.dot(h, w_ref[...], preferred_element_type=jnp.float32) + bias_ref[...]
    )


def _attn_one(q, k, v):
    s = jax.lax.dot_general(
        q, k, (((1,), (1,)), ((), ())), preferred_element_type=jnp.float32
    ) * (1.0 / 8.0)
    s = s - jnp.max(s, axis=-1, keepdims=True)
    p = jnp.exp(s)
    p = p / jnp.sum(p, axis=-1, keepdims=True)
    return jnp.dot(p, v, preferred_element_type=jnp.float32)


def _attn_body(q_ref, k_ref, v_ref, o_ref):
    q = q_ref[...]
    k = k_ref[...]
    v = v_ref[...]
    o1 = _attn_one(q[:, :DH], k[:, :DH], v[:, :DH])
    o2 = _attn_one(q[:, DH:], k[:, DH:], v[:, DH:])
    o_ref[...] = jnp.concatenate([o1, o2], axis=1)


def _proj_ln2_router_body(
    ao_ref, x_ref, wo_ref, bo_ref, g_ref, b_ref, wr_ref, x1_ref, t_ref, lg_ref
):
    o = jnp.dot(ao_ref[...], wo_ref[...], preferred_element_type=jnp.float32)
    x1 = x_ref[...] + o + bo_ref[...]
    x1_ref[...] = x1
    m = jnp.mean(x1, axis=-1, keepdims=True)
    v = jnp.mean((x1 - m) ** 2, axis=-1, keepdims=True)
    t = (x1 - m) * jax.lax.rsqrt(v + 1e-5) * g_ref[...] + b_ref[...]
    t_ref[...] = t
    lg_ref[...] = jnp.dot(t, wr_ref[...], preferred_element_type=jnp.float32)


def _routing_body(lg_ref, comb_ref, aux_ref):
    l = lg_ref[...]
    mx = jnp.max(l, axis=-1, keepdims=True)
    p = jnp.exp(l - mx)
    p = p / jnp.sum(p, axis=-1, keepdims=True)
    col = jax.lax.broadcasted_iota(jnp.int32, (S, E), 1)
    m1 = jnp.max(p, axis=-1, keepdims=True)
    i1 = jnp.min(jnp.where(p == m1, col, E), axis=-1, keepdims=True)
    pm = jnp.where(col == i1, -1.0, p)
    m2 = jnp.max(pm, axis=-1, keepdims=True)
    i2 = jnp.min(jnp.where(pm == m2, col, E), axis=-1, keepdims=True)
    den = m1 + m2
    g1 = m1 / den
    g2 = m2 / den
    oh1 = (col == i1).astype(jnp.float32)
    oh2 = (col == i2).astype(jnp.float32)
    comb_ref[...] = oh1 * g1 + oh2 * g2
    cnt = jnp.sum(oh1 + oh2, axis=0, keepdims=True)
    pk = jnp.mean(p, axis=0, keepdims=True)
    aux_ref[...] = (E / S) * jnp.sum(cnt * pk, axis=-1, keepdims=True)


def _dense_moe_body(t_ref, w1_ref, b1_ref, w2_ref, b2_ref, comb_ref, x1_ref, y_ref):
    e = pl.program_id(1)

    @pl.when(e == 0)
    def _():
        y_ref[...] = x1_ref[...]

    t = t_ref[...]
    h1 = jnp.maximum(
        jnp.dot(t, w1_ref[0], preferred_element_type=jnp.float32) + b1_ref[0], 0.0
    )
    eo = jnp.dot(h1, w2_ref[0], preferred_element_type=jnp.float32) + b2_ref[0]
    col = jax.lax.broadcasted_iota(jnp.int32, (QB, E), 1)
    w = jnp.sum(comb_ref[...] * (col == e).astype(jnp.float32), axis=-1, keepdims=True)
    y_ref[...] += w * eo


def kernel(x, ln1_g, ln1_b, ln2_g, ln2_b, Wqkv, bqkv, Wo, bo, Wr, W1, b1, W2, b2):
    x2 = x.reshape(S, D)
    g1r = ln1_g.reshape(1, D)
    b1r = ln1_b.reshape(1, D)
    g2r = ln2_g.reshape(1, D)
    b2r = ln2_b.reshape(1, D)
    bqkv_r = bqkv.reshape(1, 3 * D)
    bo_r = bo.reshape(1, D)

    qkv = pl.pallas_call(
        _ln_qkv_body,
        grid=(S // SB,),
        in_specs=[
            pl.BlockSpec((SB, D), lambda i: (i, 0)),
            pl.BlockSpec((1, D), lambda i: (0, 0)),
            pl.BlockSpec((1, D), lambda i: (0, 0)),
            pl.BlockSpec((D, 3 * D), lambda i: (0, 0)),
            pl.BlockSpec((1, 3 * D), lambda i: (0, 0)),
        ],
        out_specs=pl.BlockSpec((SB, 3 * D), lambda i: (i, 0)),
        out_shape=jax.ShapeDtypeStruct((S, 3 * D), jnp.float32),
    )(x2, g1r, b1r, Wqkv, bqkv_r)

    HP = H // 2
    ao = pl.pallas_call(
        _attn_body,
        grid=(HP, S // QB),
        in_specs=[
            pl.BlockSpec((QB, 2 * DH), lambda h, j: (j, h)),
            pl.BlockSpec((S, 2 * DH), lambda h, j: (0, HP + h)),
            pl.BlockSpec((S, 2 * DH), lambda h, j: (0, 2 * HP + h)),
        ],
        out_specs=pl.BlockSpec((QB, 2 * DH), lambda h, j: (j, h)),
        out_shape=jax.ShapeDtypeStruct((S, D), jnp.float32),
    )(qkv, qkv, qkv)

    x1, t, logits = pl.pallas_call(
        _proj_ln2_router_body,
        grid=(S // SB,),
        in_specs=[
            pl.BlockSpec((SB, D), lambda i: (i, 0)),
            pl.BlockSpec((SB, D), lambda i: (i, 0)),
            pl.BlockSpec((D, D), lambda i: (0, 0)),
            pl.BlockSpec((1, D), lambda i: (0, 0)),
            pl.BlockSpec((1, D), lambda i: (0, 0)),
            pl.BlockSpec((1, D), lambda i: (0, 0)),
            pl.BlockSpec((D, E), lambda i: (0, 0)),
        ],
        out_specs=[
            pl.BlockSpec((SB, D), lambda i: (i, 0)),
            pl.BlockSpec((SB, D), lambda i: (i, 0)),
            pl.BlockSpec((SB, E), lambda i: (i, 0)),
        ],
        out_shape=[
            jax.ShapeDtypeStruct((S, D), jnp.float32),
            jax.ShapeDtypeStruct((S, D), jnp.float32),
            jax.ShapeDtypeStruct((S, E), jnp.float32),
        ],
    )(ao, x2, Wo, bo_r, g2r, b2r, Wr)

    comb, aux11 = pl.pallas_call(
        _routing_body,
        grid=(1,),
        in_specs=[pl.BlockSpec((S, E), lambda i: (0, 0))],
        out_specs=[
            pl.BlockSpec((S, E), lambda i: (0, 0)),
            pl.BlockSpec((1, 1), lambda i: (0, 0)),
        ],
        out_shape=[
            jax.ShapeDtypeStruct((S, E), jnp.float32),
            jax.ShapeDtypeStruct((1, 1), jnp.float32),
        ],
    )(logits)

    y2 = pl.pallas_call(
        _dense_moe_body,
        grid=(S // QB, E),
        in_specs=[
            pl.BlockSpec((QB, D), lambda j, e: (j, 0)),
            pl.BlockSpec((1, D, F), lambda j, e: (e, 0, 0)),
            pl.BlockSpec((1, 1, F), lambda j, e: (e, 0, 0)),
            pl.BlockSpec((1, F, D), lambda j, e: (e, 0, 0)),
            pl.BlockSpec((1, 1, D), lambda j, e: (e, 0, 0)),
            pl.BlockSpec((QB, E), lambda j, e: (j, 0)),
            pl.BlockSpec((QB, D), lambda j, e: (j, 0)),
        ],
        out_specs=pl.BlockSpec((QB, D), lambda j, e: (j, 0)),
        out_shape=jax.ShapeDtypeStruct((S, D), jnp.float32),
    )(t, W1, b1.reshape(E, 1, F), W2, b2.reshape(E, 1, D), comb, x1)

    return y2.reshape(1, S, D), aux11.reshape(())
